# baseline (speedup 1.0000x reference)
import jax
import jax.numpy as jnp
import numpy as np
from jax import lax
from jax.experimental import pallas as pl
from jax.experimental.pallas import tpu as pltpu
from jax.experimental.pallas import tpu_sc as plsc

B, L, D = 4096, 200, 64
NC, NS = 2, 16
NW = NC * NS
BPW = B // NW
NPAIR = BPW // 2
PR = 2 * L
NGRP = PR // 16


def _body(x0_hbm, x1_hbm, out_hbm, tbl_v, idx_v, out_v):
    c = lax.axis_index("c")
    s = lax.axis_index("s")
    w = c * NS + s
    batch0 = w * BPW

    lane = lax.iota(jnp.int32, 16)
    l_vec = jnp.full((16,), L, jnp.int32)

    def pair_body(p, carry):
        b = batch0 + 2 * p
        pltpu.sync_copy(x1_hbm.at[w, p], idx_v)
        pltpu.sync_copy(x0_hbm.at[pl.ds(b, 2)], tbl_v)

        for g in range(NGRP):
            e = lane + jnp.full((16,), g * 16, jnp.int32)
            sel = lax.div(e, l_vec)
            row_out = lax.sub(e, lax.mul(sel, l_vec))
            row_tab = idx_v[pl.ds(g * 16, 16)]

            def d_body(t, carry2, sel=sel, row_out=row_out, row_tab=row_tab):
                for k in range(16):
                    dcol = jnp.full((16,), t * 16 + k, jnp.int32)
                    v = plsc.load_gather(tbl_v, [sel, row_tab, dcol])
                    plsc.store_scatter(out_v, [sel, row_out, dcol], v)
                return carry2

            lax.fori_loop(0, D // 16, d_body, 0)

        pltpu.sync_copy(out_v, out_hbm.at[pl.ds(b, 2)])
        return carry

    lax.fori_loop(0, NPAIR, pair_body, 0)


@jax.jit
def kernel(x0, x1):
    x1p = x1.astype(jnp.int32).reshape(NW, NPAIR, PR)
    mesh = plsc.VectorSubcoreMesh(core_axis_name="c", subcore_axis_name="s")
    out = pl.kernel(
        _body,
        mesh=mesh,
        out_type=jax.ShapeDtypeStruct((B, L, D), jnp.float32),
        scratch_types=[
            pltpu.VMEM((2, L, D), jnp.float32),
            pltpu.VMEM((PR,), jnp.int32),
            pltpu.VMEM((2, L, D), jnp.float32),
        ],
        compiler_params=pltpu.CompilerParams(
            use_tc_tiling_on_sc=True, needs_layout_passes=False
        ),
    )(x0, x1p)
    return out


# speedup vs baseline: 2.1243x; 2.1243x over previous
import jax
import jax.numpy as jnp
from jax import lax
from jax.experimental import pallas as pl
from jax.experimental.pallas import tpu as pltpu
from jax.experimental.pallas import tpu_sc as plsc

B, L, D = 4096, 200, 64
NC, NS = 2, 16
NW = NC * NS
BPW = B // NW
NPAIR = BPW // 2
PR = 2 * L
CHUNKS = ((0, 128), (128, 72), (200, 128), (328, 72))


def _body(x0_hbm, x1_hbm, out_hbm, scr_hbm, tbl_v, idx_v, g0, g1, g2, g3, sem):
    c = lax.axis_index("c")
    s = lax.axis_index("s")
    w = c * NS + s
    batch0 = w * BPW
    gbufs = (g0, g1, g2, g3)

    lane = lax.iota(jnp.int32, 16)
    l_vec = jnp.full((16,), L, jnp.int32)

    def pair_body(p, carry):
        b = batch0 + 2 * p
        pltpu.sync_copy(x1_hbm.at[w, p], idx_v)
        pltpu.sync_copy(x0_hbm.at[pl.ds(b, 2)], tbl_v)
        for h in range(2):
            pltpu.sync_copy(tbl_v.at[h], scr_hbm.at[pl.ds((2 * w + h) * L, L)])

        wbase = jnp.full((16,), 2 * w * L, jnp.int32)
        for g in range(PR // 16):
            e = lane + jnp.full((16,), g * 16, jnp.int32)
            off = lax.add(wbase, lax.mul(lax.div(e, l_vec), l_vec))
            sl = pl.ds(g * 16, 16)
            idx_v[sl] = idx_v[sl] + off

        for q, (o, n) in enumerate(CHUNKS):
            pltpu.async_copy(
                scr_hbm.at[idx_v.at[pl.ds(o, n)]], gbufs[q], sem
            ).wait()
        for q, (o, n) in enumerate(CHUNKS):
            pltpu.sync_copy(
                gbufs[q], out_hbm.at[b + q // 2, pl.ds(o - (q // 2) * L, n)]
            )
        return carry

    lax.fori_loop(0, NPAIR, pair_body, 0)


@jax.jit
def kernel(x0, x1):
    x1p = x1.astype(jnp.int32).reshape(NW, NPAIR, PR)
    mesh = plsc.VectorSubcoreMesh(core_axis_name="c", subcore_axis_name="s")
    out = pl.kernel(
        _body,
        mesh=mesh,
        out_type=jax.ShapeDtypeStruct((B, L, D), jnp.float32),
        scratch_types=[
            pltpu.HBM((2 * NW * L, D), jnp.float32),
            pltpu.VMEM((2, L, D), jnp.float32),
            pltpu.VMEM((PR,), jnp.int32),
            pltpu.VMEM((128, D), jnp.float32),
            pltpu.VMEM((72, D), jnp.float32),
            pltpu.VMEM((128, D), jnp.float32),
            pltpu.VMEM((72, D), jnp.float32),
            pltpu.SemaphoreType.DMA,
        ],
        compiler_params=pltpu.CompilerParams(
            use_tc_tiling_on_sc=True, needs_layout_passes=False
        ),
    )(x0, x1p)
    return out


# speedup vs baseline: 2.2168x; 1.0436x over previous
import jax
import jax.numpy as jnp
from jax import lax
from jax.experimental import pallas as pl
from jax.experimental.pallas import tpu as pltpu
from jax.experimental.pallas import tpu_sc as plsc

B, L, D = 4096, 200, 64
NC, NS = 2, 16
NW = NC * NS
ROWS = B * L
RPW = ROWS // NW
CHUNK = 128
NCHUNK = RPW // CHUNK


def _body(x0_hbm, x1_hbm, out_hbm, idx_v, rows_v, sem):
    c = lax.axis_index("c")
    s = lax.axis_index("s")
    w = c * NS + s

    pltpu.sync_copy(x1_hbm.at[w], idx_v)

    def g_body(j, carry):
        pltpu.async_copy(x0_hbm.at[idx_v.at[j]], rows_v, sem).wait()
        pltpu.sync_copy(rows_v, out_hbm.at[w, j])
        return carry

    lax.fori_loop(0, NCHUNK, g_body, 0)


@jax.jit
def kernel(x0, x1):
    x0f = x0.reshape(ROWS, D)
    x1f = (x1.astype(jnp.int32)
           + jnp.arange(B, dtype=jnp.int32)[:, None] * L)
    x1f = x1f.reshape(NW, NCHUNK, CHUNK)
    mesh = plsc.VectorSubcoreMesh(core_axis_name="c", subcore_axis_name="s")
    out = pl.kernel(
        _body,
        mesh=mesh,
        out_type=jax.ShapeDtypeStruct((NW, NCHUNK, CHUNK, D), jnp.float32),
        scratch_types=[
            pltpu.VMEM((NCHUNK, CHUNK), jnp.int32),
            pltpu.VMEM((CHUNK, D), jnp.float32),
            pltpu.SemaphoreType.DMA,
        ],
        compiler_params=pltpu.CompilerParams(use_tc_tiling_on_sc=False),
    )(x0f, x1f)
    return out.reshape(B, L, D)
